# baseline (speedup 1.0000x reference)
import functools

import jax
import jax.numpy as jnp
from jax.experimental import pallas as pl

FEATURE_DIM = 256
NUM_ATOMS = 512
BATCH = 1024
SPARSITY = 5

_TILE_B = 128


def _odl_tile_kernel(x_ref, d_ref, codes_ref, recon_ref):
    d = d_ref[:, :]
    xt = x_ref[:, :]
    norm = jnp.sqrt(jnp.sum(d * d, axis=1, keepdims=True))
    inv_norm = 1.0 / jnp.maximum(norm, 1e-12)
    corr = jnp.abs(jnp.dot(xt, d.T, preferred_element_type=jnp.float32))
    corr = corr * inv_norm.T
    cols = jax.lax.broadcasted_iota(jnp.int32, corr.shape, 1)
    m = jnp.max(corr, axis=1, keepdims=True)
    idx = jnp.min(jnp.where(corr == m, cols, NUM_ATOMS), axis=1)
    coeff_last = jnp.zeros((corr.shape[0], 1), dtype=x_ref.dtype)
    codes = jnp.where(cols == idx[:, None], coeff_last, 0.0)
    codes_ref[:, :] = codes
    recon_ref[:, :] = jnp.dot(codes * inv_norm.T, d,
                              preferred_element_type=jnp.float32)


@functools.partial(jax.jit, static_argnames=())
def kernel(x, dictionary):
    b, f = x.shape
    k = dictionary.shape[0]
    grid = (b // _TILE_B,)
    codes, recon = pl.pallas_call(
        _odl_tile_kernel,
        grid=grid,
        in_specs=[
            pl.BlockSpec((_TILE_B, f), lambda i: (i, 0)),
            pl.BlockSpec((k, f), lambda i: (0, 0)),
        ],
        out_specs=[
            pl.BlockSpec((_TILE_B, k), lambda i: (i, 0)),
            pl.BlockSpec((_TILE_B, f), lambda i: (i, 0)),
        ],
        out_shape=[
            jax.ShapeDtypeStruct((b, k), x.dtype),
            jax.ShapeDtypeStruct((b, f), x.dtype),
        ],
    )(x, dictionary)
    return codes, recon


# speedup vs baseline: 1.9202x; 1.9202x over previous
import functools

import jax
import jax.numpy as jnp
from jax.experimental import pallas as pl

FEATURE_DIM = 256
NUM_ATOMS = 512
BATCH = 1024
SPARSITY = 5

_TILE_B = 512


def _odl_tile_kernel(x_ref, d_ref, codes_ref, recon_ref):
    d = d_ref[:, :]
    xt = x_ref[:, :]
    norm = jnp.sqrt(jnp.sum(d * d, axis=1, keepdims=True))
    inv_norm = 1.0 / jnp.maximum(norm, 1e-12)
    corr = jnp.abs(jnp.dot(xt, d.T, preferred_element_type=jnp.float32))
    corr = corr * inv_norm.T
    cols = jax.lax.broadcasted_iota(jnp.int32, corr.shape, 1)
    m = jnp.max(corr, axis=1, keepdims=True)
    idx = jnp.min(jnp.where(corr == m, cols, NUM_ATOMS), axis=1)
    coeff_last = jnp.zeros((corr.shape[0], 1), dtype=x_ref.dtype)
    codes = jnp.where(cols == idx[:, None], coeff_last, 0.0)
    codes_ref[:, :] = codes
    recon_ref[:, :] = jnp.dot(codes * inv_norm.T, d,
                              preferred_element_type=jnp.float32)


@functools.partial(jax.jit, static_argnames=())
def kernel(x, dictionary):
    b, f = x.shape
    k = dictionary.shape[0]
    grid = (b // _TILE_B,)
    codes, recon = pl.pallas_call(
        _odl_tile_kernel,
        grid=grid,
        in_specs=[
            pl.BlockSpec((_TILE_B, f), lambda i: (i, 0)),
            pl.BlockSpec((k, f), lambda i: (0, 0)),
        ],
        out_specs=[
            pl.BlockSpec((_TILE_B, k), lambda i: (i, 0)),
            pl.BlockSpec((_TILE_B, f), lambda i: (i, 0)),
        ],
        out_shape=[
            jax.ShapeDtypeStruct((b, k), x.dtype),
            jax.ShapeDtypeStruct((b, f), x.dtype),
        ],
    )(x, dictionary)
    return codes, recon
